# baseline (speedup 1.0000x reference)
import functools

import jax
import jax.numpy as jnp
from jax import lax
from jax.experimental import pallas as pl
from jax.experimental.pallas import tpu as pltpu
from jax.experimental.pallas import tpu_sc as plsc

DIM = 64


@functools.lru_cache(maxsize=None)
def _make_gather(N: int, S: int):
    info = plsc.get_sparse_core_info()
    NC, NS, L = info.num_cores, info.num_subcores, info.num_lanes
    NW = NC * NS
    NB = N // 128
    assert NB == NW and S % 2 == 0 and L == 16
    PAIRS = S // 2
    T = S * 8 * NB

    mesh = plsc.VectorSubcoreMesh(core_axis_name="c", subcore_axis_name="s")

    @functools.partial(
        pl.kernel,
        mesh=mesh,
        out_type=jax.ShapeDtypeStruct((T, 1024), jnp.float32),
        compiler_params=pltpu.CompilerParams(
            use_tc_tiling_on_sc=False, needs_layout_passes=False
        ),
        scratch_types=[
            pltpu.VMEM((S, 128), jnp.int32),
            pltpu.VMEM((PAIRS, 16), jnp.int32),
            pltpu.VMEM((2, 2, 128, DIM), jnp.float32),
            pltpu.VMEM((2, 16, 1024), jnp.float32),
            pltpu.SemaphoreType.DMA,
            pltpu.SemaphoreType.DMA,
            pltpu.SemaphoreType.DMA,
            pltpu.SemaphoreType.DMA,
        ],
    )
    def gather_kernel(idx_hbm, table_hbm, out_hbm, idx_v, tidx_v, rows_v,
                      tiles_v, g0, g1, s0, s1):
        gsems = (g0, g1)
        ssems = (s0, s1)
        wid = lax.axis_index("s") * NC + lax.axis_index("c")

        pltpu.sync_copy(idx_hbm.at[:, pl.ds(wid * 128, 128)], idx_v)

        iota = lax.iota(jnp.int32, L)
        iota32 = iota * 32

        def mk_tidx(k, carry):
            tidx_v[k, :] = iota32 + (k * 512 + wid)
            return carry

        lax.fori_loop(0, PAIRS, mk_tidx, 0)

        def gather_start(k, pb):
            for sb in range(2):
                pltpu.async_copy(
                    table_hbm.at[idx_v.at[2 * k + sb]],
                    rows_v.at[pb, sb],
                    gsems[pb],
                )

        def gather_wait(pb):
            for sb in range(2):
                pltpu.make_async_copy(
                    table_hbm.at[idx_v.at[0]], rows_v.at[pb, sb], gsems[pb]
                ).wait()

        def scatter_start(k, pb):
            pltpu.async_copy(
                tiles_v.at[pb], out_hbm.at[tidx_v.at[k]], ssems[pb]
            )

        def scatter_wait(pb):
            pltpu.make_async_copy(
                tiles_v.at[pb], out_hbm.at[tidx_v.at[0]], ssems[pb]
            ).wait()

        nvecs = [iota + ng * L for ng in range(8)]
        dvecs = [jnp.full((L,), d, jnp.int32) for d in range(DIM)]

        def transpose_pair(pb):
            for sb in range(2):
                src = rows_v.at[pb, sb]
                for d in range(DIM):
                    t8 = sb * 8 + d // 8
                    off = (d % 8) * 128
                    for ng in range(8):
                        v = plsc.load_gather(src, [nvecs[ng], dvecs[d]])
                        tiles_v[pb, t8, pl.ds(off + ng * L, L)] = v

        gather_start(0, 0)

        def body(g, carry):
            for pb in range(2):
                k = g + pb

                @pl.when(k >= 2)
                def _():
                    scatter_wait(pb)

                gather_wait(pb)

                @pl.when(k + 1 < PAIRS)
                def _():
                    gather_start(k + 1, 1 - pb)

                transpose_pair(pb)
                scatter_start(k, pb)
            return carry

        lax.fori_loop(0, PAIRS // 2, lambda i, c: body(i * 2, c), 0)
        scatter_wait(0)
        scatter_wait(1)

    return gather_kernel


def kernel(x, table):
    rows, cols = x.shape
    V = table.shape[0]
    xt = x.T.astype(jnp.int32)
    t2 = lax.optimization_barrier(table.reshape(V // 2, 2 * DIM))
    t3 = t2.reshape(V, DIM)
    out2 = _make_gather(rows, cols)(xt, t3)
    out5 = out2.reshape(cols, 8, rows // 128, 8, 128)
    return out5.transpose(2, 4, 0, 1, 3).reshape(rows, cols, DIM)


# speedup vs baseline: 2.2028x; 2.2028x over previous
import functools

import jax
import jax.numpy as jnp
from jax import lax
from jax.experimental import pallas as pl
from jax.experimental.pallas import tpu as pltpu
from jax.experimental.pallas import tpu_sc as plsc

DIM = 64


@functools.lru_cache(maxsize=None)
def _make_gather(B: int, C: int):
    info = plsc.get_sparse_core_info()
    NC, NS = info.num_cores, info.num_subcores
    NW = NC * NS
    n_per_w = B // NW
    steps = n_per_w // C
    assert steps * C == n_per_w and n_per_w * NW == B and steps % 2 == 0
    mesh = plsc.VectorSubcoreMesh(core_axis_name="c", subcore_axis_name="s")

    @functools.partial(
        pl.kernel,
        mesh=mesh,
        out_type=jax.ShapeDtypeStruct((B, DIM), jnp.float32),
        compiler_params=pltpu.CompilerParams(use_tc_tiling_on_sc=False),
        scratch_types=[
            pltpu.VMEM((n_per_w,), jnp.int32),
            pltpu.VMEM((2, C, DIM), jnp.float32),
            pltpu.SemaphoreType.DMA,
            pltpu.SemaphoreType.DMA,
            pltpu.SemaphoreType.DMA,
            pltpu.SemaphoreType.DMA,
        ],
    )
    def gather_kernel(idx_hbm, table_hbm, out_hbm, idx_v, rows_v, g0, g1, w0, w1):
        gsems = (g0, g1)
        wsems = (w0, w1)
        wid = lax.axis_index("s") * NC + lax.axis_index("c")
        base = wid * n_per_w

        pltpu.sync_copy(idx_hbm.at[pl.ds(base, n_per_w)], idx_v)

        def gather_start(cur, b):
            pltpu.async_copy(
                table_hbm.at[idx_v.at[pl.ds(cur * C, C)]], rows_v.at[b], gsems[b]
            )

        def gather_wait(b):
            pltpu.make_async_copy(
                table_hbm.at[idx_v.at[pl.ds(0, C)]], rows_v.at[b], gsems[b]
            ).wait()

        def wb_start(cur, b):
            pltpu.async_copy(
                rows_v.at[b], out_hbm.at[pl.ds(base + cur * C, C)], wsems[b]
            )

        def wb_wait(b):
            pltpu.make_async_copy(
                rows_v.at[b], out_hbm.at[pl.ds(base, C)], wsems[b]
            ).wait()

        gather_start(0, 0)

        def body(g, carry):
            for b in range(2):
                cur = g + b

                @pl.when(cur >= 1)
                def _():
                    wb_wait(1 - b)

                gather_wait(b)

                @pl.when(cur + 1 < steps)
                def _():
                    gather_start(cur + 1, 1 - b)

                wb_start(cur, b)
            return carry

        lax.fori_loop(0, steps // 2, lambda i, c: body(i * 2, c), 0)
        wb_wait((steps - 1) % 2)

    return gather_kernel


def kernel(x, table):
    rows, cols = x.shape
    B = rows * cols
    V = table.shape[0]
    xf = x.reshape(B).astype(jnp.int32)
    t2 = lax.optimization_barrier(table.reshape(V // 2, 2 * DIM))
    t3 = t2.reshape(V, DIM)
    out = _make_gather(B, 512)(xf, t3)
    return out.reshape(rows, cols, DIM)


# speedup vs baseline: 2.2047x; 1.0009x over previous
import functools

import jax
import jax.numpy as jnp
from jax import lax
from jax.experimental import pallas as pl
from jax.experimental.pallas import tpu as pltpu
from jax.experimental.pallas import tpu_sc as plsc

DIM = 64


@functools.lru_cache(maxsize=None)
def _make_gather(B: int, C: int):
    info = plsc.get_sparse_core_info()
    NC, NS = info.num_cores, info.num_subcores
    NW = NC * NS
    n_per_w = B // NW
    steps = n_per_w // C
    assert steps * C == n_per_w and n_per_w * NW == B and steps % 2 == 0
    mesh = plsc.VectorSubcoreMesh(core_axis_name="c", subcore_axis_name="s")

    @functools.partial(
        pl.kernel,
        mesh=mesh,
        out_type=jax.ShapeDtypeStruct((B, DIM), jnp.float32),
        compiler_params=pltpu.CompilerParams(use_tc_tiling_on_sc=False),
        scratch_types=[
            pltpu.VMEM((n_per_w,), jnp.int32),
            pltpu.VMEM((2, C, DIM), jnp.float32),
            pltpu.SemaphoreType.DMA,
            pltpu.SemaphoreType.DMA,
            pltpu.SemaphoreType.DMA,
            pltpu.SemaphoreType.DMA,
        ],
    )
    def gather_kernel(idx_hbm, table_hbm, out_hbm, idx_v, rows_v, g0, g1, w0, w1):
        gsems = (g0, g1)
        wsems = (w0, w1)
        wid = lax.axis_index("s") * NC + lax.axis_index("c")
        base = wid * n_per_w

        pltpu.sync_copy(idx_hbm.at[pl.ds(base, n_per_w)], idx_v)

        def gather_start(cur, b):
            pltpu.async_copy(
                table_hbm.at[idx_v.at[pl.ds(cur * C, C)]], rows_v.at[b], gsems[b]
            )

        def gather_wait(b):
            pltpu.make_async_copy(
                table_hbm.at[idx_v.at[pl.ds(0, C)]], rows_v.at[b], gsems[b]
            ).wait()

        def wb_start(cur, b):
            pltpu.async_copy(
                rows_v.at[b], out_hbm.at[pl.ds(base + cur * C, C)], wsems[b]
            )

        def wb_wait(b):
            pltpu.make_async_copy(
                rows_v.at[b], out_hbm.at[pl.ds(base, C)], wsems[b]
            ).wait()

        gather_start(0, 0)

        def body(g, carry):
            for b in range(2):
                cur = g + b

                @pl.when(cur >= 1)
                def _():
                    wb_wait(1 - b)

                gather_wait(b)

                @pl.when(cur + 1 < steps)
                def _():
                    gather_start(cur + 1, 1 - b)

                wb_start(cur, b)
            return carry

        lax.fori_loop(0, steps // 2, lambda i, c: body(i * 2, c), 0)
        wb_wait((steps - 1) % 2)

    return gather_kernel


def kernel(x, table):
    rows, cols = x.shape
    B = rows * cols
    V = table.shape[0]
    xf = x.reshape(B).astype(jnp.int32)
    t2 = lax.optimization_barrier(table.reshape(V // 2, 2 * DIM))
    t3 = t2.reshape(V, DIM)
    out = _make_gather(B, 800)(xf, t3)
    return out.reshape(rows, cols, DIM)


# speedup vs baseline: 2.2061x; 1.0006x over previous
import functools

import jax
import jax.numpy as jnp
from jax import lax
from jax.experimental import pallas as pl
from jax.experimental.pallas import tpu as pltpu
from jax.experimental.pallas import tpu_sc as plsc

DIM = 64


@functools.lru_cache(maxsize=None)
def _make_gather(B: int, C: int):
    info = plsc.get_sparse_core_info()
    NC, NS = info.num_cores, info.num_subcores
    NW = NC * NS
    n_per_w = B // NW
    steps = n_per_w // C
    assert steps * C == n_per_w and n_per_w * NW == B and steps % 2 == 0
    mesh = plsc.VectorSubcoreMesh(core_axis_name="c", subcore_axis_name="s")

    @functools.partial(
        pl.kernel,
        mesh=mesh,
        out_type=jax.ShapeDtypeStruct((B, DIM), jnp.float32),
        compiler_params=pltpu.CompilerParams(use_tc_tiling_on_sc=False),
        scratch_types=[
            pltpu.VMEM((n_per_w,), jnp.int32),
            pltpu.VMEM((2, C, DIM), jnp.float32),
            pltpu.SemaphoreType.DMA,
            pltpu.SemaphoreType.DMA,
            pltpu.SemaphoreType.DMA,
            pltpu.SemaphoreType.DMA,
        ],
    )
    def gather_kernel(idx_hbm, table_hbm, out_hbm, idx_v, rows_v, g0, g1, w0, w1):
        gsems = (g0, g1)
        wsems = (w0, w1)
        wid = lax.axis_index("s") * NC + lax.axis_index("c")
        base = wid * n_per_w

        pltpu.sync_copy(idx_hbm.at[pl.ds(base, n_per_w)], idx_v)

        def gather_start(cur, b):
            pltpu.async_copy(
                table_hbm.at[idx_v.at[pl.ds(cur * C, C)]], rows_v.at[b], gsems[b]
            )

        def gather_wait(b):
            pltpu.make_async_copy(
                table_hbm.at[idx_v.at[pl.ds(0, C)]], rows_v.at[b], gsems[b]
            ).wait()

        def wb_start(cur, b):
            pltpu.async_copy(
                rows_v.at[b], out_hbm.at[pl.ds(base + cur * C, C)], wsems[b]
            )

        def wb_wait(b):
            pltpu.make_async_copy(
                rows_v.at[b], out_hbm.at[pl.ds(base, C)], wsems[b]
            ).wait()

        gather_start(0, 0)

        def body(g, carry):
            for b in range(2):
                cur = g + b

                @pl.when(cur >= 1)
                def _():
                    wb_wait(1 - b)

                gather_wait(b)

                @pl.when(cur + 1 < steps)
                def _():
                    gather_start(cur + 1, 1 - b)

                wb_start(cur, b)
            return carry

        lax.fori_loop(0, steps // 2, lambda i, c: body(i * 2, c), 0)
        wb_wait((steps - 1) % 2)

    return gather_kernel


def kernel(x, table):
    rows, cols = x.shape
    B = rows * cols
    V = table.shape[0]
    xf = x.reshape(B).astype(jnp.int32)
    t2 = lax.optimization_barrier(table.reshape(V // 2, 2 * DIM))
    t3 = t2.reshape(V, DIM)
    out = _make_gather(B, 512)(xf, t3)
    return out.reshape(rows, cols, DIM)
